# Initial kernel scaffold; baseline (speedup 1.0000x reference)
import jax
import jax.numpy as jnp
from jax import lax
from jax.experimental import pallas as pl
from jax.experimental.pallas import tpu as pltpu
from jax.experimental.pallas import tpu_sc as plsc

EMB = 32
NC = 2
NS = 16
NW = NC * NS
SUB = 128
CH = 8


def _gather_call(n_rows, table, idx2d):
    assert n_rows % (NW * SUB * CH) == 0
    rows_per_w = n_rows // NW
    sub_per_w = rows_per_w // SUB
    n_chunks = sub_per_w // CH

    mesh = plsc.VectorSubcoreMesh(
        core_axis_name="c", subcore_axis_name="s", num_cores=NC,
        num_subcores=NS)

    @pl.kernel(
        out_type=jax.ShapeDtypeStruct((n_rows, EMB), jnp.float32),
        mesh=mesh,
        scratch_types=[
            pltpu.VMEM((CH, SUB), jnp.int32),
            pltpu.VMEM((CH * SUB, EMB), jnp.float32),
            pltpu.SemaphoreType.DMA,
        ],
    )
    def k(table_hbm, idx_hbm, out_hbm, idx_v, rows_v, sem):
        wid = lax.axis_index("s") * NC + lax.axis_index("c")
        row0 = wid * sub_per_w

        def chunk(c):
            r = row0 + c * CH
            pltpu.sync_copy(idx_hbm.at[pl.ds(r, CH), :], idx_v)
            copies = [
                pltpu.async_copy(
                    table_hbm.at[idx_v.at[j]],
                    rows_v.at[pl.ds(j * SUB, SUB), :],
                    sem,
                )
                for j in range(CH)
            ]
            for cp in copies:
                cp.wait()
            pltpu.sync_copy(rows_v, out_hbm.at[pl.ds(r * SUB, CH * SUB), :])

        pl.loop(0, n_chunks)(chunk)

    return k(table, idx2d)


def kernel(inputs, table):
    B, L = inputs.shape
    n = B * L
    idx = inputs.reshape(-1).astype(jnp.int32)
    idx2d = idx.reshape(n // SUB, SUB)
    out = _gather_call(n, table, idx2d)
    return out.reshape(B, L, EMB)


# speedup vs baseline: 1.0952x; 1.0952x over previous
import jax
import jax.numpy as jnp
from jax import lax
from jax.experimental import pallas as pl
from jax.experimental.pallas import tpu as pltpu
from jax.experimental.pallas import tpu_sc as plsc

EMB = 32
NC = 2
NS = 16
NW = NC * NS
SUB = 128
CH = 8


def _gather_call(n_rows, table, idx2d):
    assert n_rows % (NW * SUB * CH) == 0
    rows_per_w = n_rows // NW
    sub_per_w = rows_per_w // SUB
    n_chunks = sub_per_w // CH

    mesh = plsc.VectorSubcoreMesh(
        core_axis_name="c", subcore_axis_name="s", num_cores=NC,
        num_subcores=NS)

    @pl.kernel(
        out_type=jax.ShapeDtypeStruct((n_rows, EMB), jnp.float32),
        mesh=mesh,
        compiler_params=pltpu.CompilerParams(use_tc_tiling_on_sc=False),
        scratch_types=[
            pltpu.VMEM((CH, SUB), jnp.int32),
            pltpu.VMEM((CH * SUB, EMB), jnp.float32),
            pltpu.SemaphoreType.DMA,
        ],
    )
    def k(table_hbm, idx_hbm, out_hbm, idx_v, rows_v, sem):
        wid = lax.axis_index("s") * NC + lax.axis_index("c")
        row0 = wid * sub_per_w

        def chunk(c):
            r = row0 + c * CH
            pltpu.sync_copy(idx_hbm.at[pl.ds(r, CH), :], idx_v)
            copies = [
                pltpu.async_copy(
                    table_hbm.at[idx_v.at[j]],
                    rows_v.at[pl.ds(j * SUB, SUB), :],
                    sem,
                )
                for j in range(CH)
            ]
            for cp in copies:
                cp.wait()
            pltpu.sync_copy(rows_v, out_hbm.at[pl.ds(r * SUB, CH * SUB), :])

        pl.loop(0, n_chunks)(chunk)

    return k(table, idx2d)


def kernel(inputs, table):
    B, L = inputs.shape
    n = B * L
    idx = inputs.reshape(-1).astype(jnp.int32)
    idx2d = idx.reshape(n // SUB, SUB)
    out = _gather_call(n, table, idx2d)
    return out.reshape(B, L, EMB)


# speedup vs baseline: 1.1135x; 1.0167x over previous
import jax
import jax.numpy as jnp
from jax import lax
from jax.experimental import pallas as pl
from jax.experimental.pallas import tpu as pltpu
from jax.experimental.pallas import tpu_sc as plsc

EMB = 32
NC = 2
NS = 16
NW = NC * NS
SUB = 128
CH = 10


def _gather_call(n_rows, table, idx2d):
    assert n_rows % (NW * SUB * CH * 2) == 0
    rows_per_w = n_rows // NW
    sub_per_w = rows_per_w // SUB
    n_chunks = sub_per_w // CH
    n_pairs = n_chunks // 2

    mesh = plsc.VectorSubcoreMesh(
        core_axis_name="c", subcore_axis_name="s", num_cores=NC,
        num_subcores=NS)

    @pl.kernel(
        out_type=jax.ShapeDtypeStruct((n_rows, EMB), jnp.float32),
        mesh=mesh,
        compiler_params=pltpu.CompilerParams(use_tc_tiling_on_sc=False),
        scratch_types=[
            pltpu.VMEM((sub_per_w, SUB), jnp.int32),
            pltpu.VMEM((2, CH * SUB, EMB), jnp.float32),
            pltpu.SemaphoreType.DMA,
            pltpu.SemaphoreType.DMA,
            pltpu.SemaphoreType.DMA,
            pltpu.SemaphoreType.DMA,
        ],
    )
    def k(table_hbm, idx_hbm, out_hbm, idx_v, rows_v, g0, g1, o0, o1):
        wid = lax.axis_index("s") * NC + lax.axis_index("c")
        row0 = wid * sub_per_w
        out0 = row0 * SUB
        sems_g = (g0, g1)
        sems_o = (o0, o1)

        pltpu.sync_copy(idx_hbm.at[pl.ds(row0, sub_per_w), :], idx_v)

        def fire_gathers(c, b):
            for j in range(CH):
                pltpu.async_copy(
                    table_hbm.at[idx_v.at[c * CH + j]],
                    rows_v.at[b].at[pl.ds(j * SUB, SUB), :],
                    sems_g[b],
                )

        def wait_gathers(b):
            for j in range(CH):
                pltpu.make_async_copy(
                    table_hbm.at[idx_v.at[j]],
                    rows_v.at[b].at[pl.ds(j * SUB, SUB), :],
                    sems_g[b],
                ).wait()

        def fire_out(c, b):
            pltpu.async_copy(
                rows_v.at[b],
                out_hbm.at[pl.ds(out0 + c * (CH * SUB), CH * SUB), :],
                sems_o[b],
            )

        def wait_out(b):
            pltpu.make_async_copy(
                rows_v.at[b],
                out_hbm.at[pl.ds(out0, CH * SUB), :],
                sems_o[b],
            ).wait()

        fire_gathers(0, 0)

        def pair(h):
            c0 = h * 2
            pl.when(h > 0)(lambda: wait_out(1))
            fire_gathers(c0 + 1, 1)
            wait_gathers(0)
            fire_out(c0, 0)

            def prefetch_next():
                wait_out(0)
                fire_gathers(c0 + 2, 0)
            pl.when(h < n_pairs - 1)(prefetch_next)
            wait_gathers(1)
            fire_out(c0 + 1, 1)

        pl.loop(0, n_pairs)(pair)
        wait_out(0)
        wait_out(1)

    return k(table, idx2d)


def kernel(inputs, table):
    B, L = inputs.shape
    n = B * L
    idx = inputs.reshape(-1).astype(jnp.int32)
    idx2d = idx.reshape(n // SUB, SUB)
    out = _gather_call(n, table, idx2d)
    return out.reshape(B, L, EMB)


# speedup vs baseline: 1.7931x; 1.6104x over previous
import jax
import jax.numpy as jnp
from jax import lax
from jax.experimental import pallas as pl
from jax.experimental.pallas import tpu as pltpu
from jax.experimental.pallas import tpu_sc as plsc

EMB = 32
NC = 2
NS = 16
NW = NC * NS
SUB = 128
CB = 64


def _gather_call(B, L, idx2d, table):
    n_rows = B * L
    b_per_w = B // NW
    rows_per_w = b_per_w * L
    sub_per_w = rows_per_w // SUB
    rows_per_ch = CB * L
    sub_per_ch = rows_per_ch // SUB
    n_chunks = b_per_w // CB
    half_rows = rows_per_ch // 2
    assert half_rows == (CB // 2) * L
    sub_a = (half_rows + SUB - 1) // SUB

    mesh = plsc.VectorSubcoreMesh(
        core_axis_name="c", subcore_axis_name="s", num_cores=NC,
        num_subcores=NS)

    @pl.kernel(
        out_type=jax.ShapeDtypeStruct((B, L, EMB), jnp.float32),
        mesh=mesh,
        compiler_params=pltpu.CompilerParams(use_tc_tiling_on_sc=False),
        scratch_types=[
            pltpu.VMEM((sub_per_w, SUB), jnp.int32),
            pltpu.VMEM((rows_per_ch, EMB), jnp.float32),
            pltpu.SemaphoreType.DMA,
            pltpu.SemaphoreType.DMA,
            pltpu.SemaphoreType.DMA,
        ],
    )
    def k(idx_hbm, table_hbm, out_hbm, idx_v, rows_v, sg, soa, sob):
        wid = lax.axis_index("s") * NC + lax.axis_index("c")
        batch0 = wid * b_per_w
        row0 = wid * sub_per_w

        pltpu.sync_copy(idx_hbm.at[pl.ds(row0, sub_per_w), :], idx_v)

        def fire_gathers(c, j0, j1):
            for j in range(j0, j1):
                pltpu.async_copy(
                    table_hbm.at[idx_v.at[c * sub_per_ch + j]],
                    rows_v.at[pl.ds(j * SUB, SUB), :],
                    sg,
                )

        def wait_gathers(j0, j1):
            for j in range(j0, j1):
                pltpu.make_async_copy(
                    table_hbm.at[idx_v.at[j]],
                    rows_v.at[pl.ds(j * SUB, SUB), :],
                    sg,
                ).wait()

        def fire_outs(c, h, sem):
            def one(bb):
                pltpu.async_copy(
                    rows_v.at[pl.ds(h * half_rows + bb * L, L), :],
                    out_hbm.at[batch0 + c * CB + h * (CB // 2) + bb],
                    sem,
                )
            pl.loop(h * (CB // 2), (h + 1) * (CB // 2))(
                lambda bb: one(bb - h * (CB // 2)))

        def wait_outs(sem):
            def one(bb):
                pltpu.make_async_copy(
                    rows_v.at[pl.ds(0, L), :],
                    out_hbm.at[batch0],
                    sem,
                ).wait()
            pl.loop(0, CB // 2)(one)

        def chunk(c):
            def drain_prev():
                wait_outs(soa)
                wait_outs(sob)
            pl.when(c > 0)(drain_prev)
            fire_gathers(c, 0, sub_a)
            wait_gathers(0, sub_a)
            fire_outs(c, 0, soa)
            fire_gathers(c, sub_a, sub_per_ch)
            wait_gathers(sub_a, sub_per_ch)
            fire_outs(c, 1, sob)

        pl.loop(0, n_chunks)(chunk)
        wait_outs(soa)
        wait_outs(sob)

    return k(idx2d, table)


def kernel(inputs, table):
    B, L = inputs.shape
    idx = inputs.reshape(-1)
    if idx.dtype != jnp.int32:
        idx = idx.astype(jnp.int32)
    idx2d = idx.reshape((B * L) // SUB, SUB)
    return _gather_call(B, L, idx2d, table)


# speedup vs baseline: 2.5163x; 1.4033x over previous
import jax
import jax.numpy as jnp
from jax import lax
from jax.experimental import pallas as pl
from jax.experimental.pallas import tpu as pltpu
from jax.experimental.pallas import tpu_sc as plsc

EMB = 32
NC = 2
NS = 16
NW = NC * NS
SUB = 128
CB = 64


def _gather_call(B, L, idx2d, table):
    n_rows = B * L
    b_per_w = B // NW
    rows_per_w = b_per_w * L
    sub_per_w = rows_per_w // SUB
    rows_per_ch = CB * L
    sub_per_ch = rows_per_ch // SUB
    n_chunks = b_per_w // CB
    half_rows = rows_per_ch // 2
    assert half_rows == (CB // 2) * L
    sub_a = (half_rows + SUB - 1) // SUB

    mesh = plsc.VectorSubcoreMesh(
        core_axis_name="c", subcore_axis_name="s", num_cores=NC,
        num_subcores=NS)

    LP = (L + 7) // 8 * 8
    MP = 128

    @pl.kernel(
        out_type=jax.ShapeDtypeStruct((B, LP, MP), jnp.float32),
        mesh=mesh,
        compiler_params=pltpu.CompilerParams(use_tc_tiling_on_sc=False),
        scratch_types=[
            pltpu.VMEM((sub_per_w, SUB), jnp.int32),
            pltpu.VMEM((rows_per_ch, EMB), jnp.float32),
            pltpu.SemaphoreType.DMA,
            pltpu.SemaphoreType.DMA,
            pltpu.SemaphoreType.DMA,
        ],
    )
    def k(idx_hbm, table_hbm, out_hbm, idx_v, rows_v, sg, soa, sob):
        wid = lax.axis_index("s") * NC + lax.axis_index("c")
        batch0 = wid * b_per_w
        row0 = wid * sub_per_w

        pltpu.sync_copy(idx_hbm.at[pl.ds(row0, sub_per_w), :], idx_v)

        def fire_gathers(c, j0, j1):
            for j in range(j0, j1):
                pltpu.async_copy(
                    table_hbm.at[idx_v.at[c * sub_per_ch + j]],
                    rows_v.at[pl.ds(j * SUB, SUB), :],
                    sg,
                )

        def wait_gathers(j0, j1):
            for j in range(j0, j1):
                pltpu.make_async_copy(
                    table_hbm.at[idx_v.at[j]],
                    rows_v.at[pl.ds(j * SUB, SUB), :],
                    sg,
                ).wait()

        def fire_outs(c, h, sem):
            def one(bb):
                pltpu.async_copy(
                    rows_v.at[pl.ds(h * half_rows + bb * L, L), :],
                    out_hbm.at[batch0 + c * CB + h * (CB // 2) + bb,
                               pl.ds(0, L), pl.ds(0, EMB)],
                    sem,
                )
            pl.loop(h * (CB // 2), (h + 1) * (CB // 2))(
                lambda bb: one(bb - h * (CB // 2)))

        def wait_outs(sem):
            def one(bb):
                pltpu.make_async_copy(
                    rows_v.at[pl.ds(0, L), :],
                    out_hbm.at[batch0, pl.ds(0, L), pl.ds(0, EMB)],
                    sem,
                ).wait()
            pl.loop(0, CB // 2)(one)

        def chunk(c):
            def drain_prev():
                wait_outs(soa)
                wait_outs(sob)
            pl.when(c > 0)(drain_prev)
            fire_gathers(c, 0, sub_a)
            wait_gathers(0, sub_a)
            fire_outs(c, 0, soa)
            fire_gathers(c, sub_a, sub_per_ch)
            wait_gathers(sub_a, sub_per_ch)
            fire_outs(c, 1, sob)

        pl.loop(0, n_chunks)(chunk)
        wait_outs(soa)
        wait_outs(sob)

    out_padded = k(idx2d, table)
    return out_padded[:, :L, :EMB]


def kernel(inputs, table):
    B, L = inputs.shape
    idx = inputs.reshape(-1)
    if idx.dtype != jnp.int32:
        idx = idx.astype(jnp.int32)
    idx2d = idx.reshape((B * L) // SUB, SUB)
    return _gather_call(B, L, idx2d, table)
